# baseline (speedup 1.0000x reference)
import jax
import jax.numpy as jnp
from jax import lax
from jax.experimental import pallas as pl
from jax.experimental.pallas import tpu as pltpu
from jax.experimental.pallas import tpu_sc as plsc

N = 10000
B = 128
E = 320000
NC, NS, L = 2, 16, 16
W = NC * NS
BLK = 128
BPW = 80
NBLK = BPW * W
E_PAD = NBLK * BLK
STRIPE = 624
STRIPE_LAST = N - STRIPE * (NS - 1)
HALF = 40
GRID = 10


def _exp_body(x_ref, o_ref):
    o_ref[...] = jnp.exp(x_ref[...])


def _log_body(p_ref, o_ref):
    s = p_ref[0] + p_ref[1]
    o_ref[...] = jnp.log(jnp.maximum(s, 1e-30))


def _sc_body(ev, meta, pf, zeros, out, meta_v, p_v, rows_v, src_sv, dst_sv,
             s_sh, gsem, msem):
    cid = lax.axis_index("c")
    sid = lax.axis_index("s")
    wid = cid * NS + sid

    pltpu.async_copy(meta.at[pl.ds(wid * BPW, HALF)], meta_v, msem)
    pltpu.async_copy(pf.at[pl.ds(wid * BPW, HALF)], p_v, msem)

    r0 = sid * STRIPE

    @pl.when(sid < NS - 1)
    def _():
        pltpu.sync_copy(zeros.at[pl.ds(r0, STRIPE)],
                        s_sh.at[pl.ds(r0, STRIPE)])

    @pl.when(sid == NS - 1)
    def _():
        pltpu.sync_copy(zeros.at[pl.ds(r0, STRIPE_LAST)],
                        s_sh.at[pl.ds(r0, STRIPE_LAST)])

    pltpu.make_async_copy(meta.at[pl.ds(0, HALF)], meta_v, msem).wait()
    pltpu.make_async_copy(pf.at[pl.ds(0, HALF)], p_v, msem).wait()

    plsc.subcore_barrier()

    for phase in range(2):
        def outer(t, carry):
            b = 0

            for g in range(BLK // L):
                src_sv[pl.ds(g * L, L)] = meta_v[t, 0, pl.ds(g * L, L)]
                dst_sv[pl.ds(g * L, L)] = meta_v[t, 1, pl.ds(g * L, L)]

            pltpu.async_copy(ev.at[src_sv], rows_v.at[b], gsem).wait()

            def mul_group(g, c):
                p16 = p_v[t, 0, pl.ds(g * L, L)]
                for k in range(L):
                    ps = jnp.full((L,), p16[k], jnp.float32)
                    row = g * L + k
                    for j in range(B // L):
                        sl = (b, row, pl.ds(j * L, L))
                        rows_v[sl] = rows_v[sl] * ps
                return c

            lax.fori_loop(0, BLK // L, mul_group, 0)

            pltpu.sync_copy(rows_v.at[b], s_sh.at[dst_sv], add=True)
            return carry

        lax.fori_loop(0, HALF, outer, 0)

        if phase == 0:
            pltpu.sync_copy(meta.at[pl.ds(wid * BPW + HALF, HALF)], meta_v)
            pltpu.sync_copy(pf.at[pl.ds(wid * BPW + HALF, HALF)], p_v)

    plsc.subcore_barrier()

    @pl.when(sid < NS - 1)
    def _():
        pltpu.sync_copy(s_sh.at[pl.ds(r0, STRIPE)],
                        out.at[cid, pl.ds(r0, STRIPE)])

    @pl.when(sid == NS - 1)
    def _():
        pltpu.sync_copy(s_sh.at[pl.ds(r0, STRIPE_LAST)],
                        out.at[cid, pl.ds(r0, STRIPE_LAST)])


def kernel(ch_vals, edge_src, edge_dst, params):
    ev = pl.pallas_call(
        _exp_body,
        grid=(GRID,),
        in_specs=[pl.BlockSpec((N // GRID, B), lambda i: (i, 0))],
        out_specs=pl.BlockSpec((N // GRID, B), lambda i: (i, 0)),
        out_shape=jax.ShapeDtypeStruct((N, B), jnp.float32),
    )(ch_vals)

    pad = E_PAD - E
    zpad = jnp.zeros((pad,), jnp.int32)
    src_p = jnp.concatenate([edge_src, zpad]).reshape(BPW, W, BLK)
    dst_p = jnp.concatenate([edge_dst, zpad]).reshape(BPW, W, BLK)
    meta = jnp.stack([src_p, dst_p], axis=2).transpose(1, 0, 2, 3)
    meta = meta.reshape(W * BPW, 2, BLK)
    pf = jnp.concatenate([params, jnp.zeros((pad,), jnp.float32)]
                         ).reshape(BPW, W, 1, BLK).transpose(1, 0, 2, 3)
    pf = pf.reshape(W * BPW, 1, BLK)
    zeros = jnp.zeros((N, B), jnp.float32)

    sc = pl.kernel(
        _sc_body,
        out_type=jax.ShapeDtypeStruct((NC, N, B), jnp.float32),
        mesh=plsc.VectorSubcoreMesh(core_axis_name="c", subcore_axis_name="s"),
        scratch_types=[
            pltpu.VMEM((HALF, 2, BLK), jnp.int32),
            pltpu.VMEM((HALF, 1, BLK), jnp.float32),
            pltpu.VMEM((2, BLK, B), jnp.float32),
            pltpu.VMEM((BLK,), jnp.int32),
            pltpu.VMEM((BLK,), jnp.int32),
            pltpu.VMEM_SHARED((N, B), jnp.float32),
            pltpu.SemaphoreType.DMA,
            pltpu.SemaphoreType.DMA,
        ],
    )
    partial = sc(ev, meta, pf, zeros)

    out = pl.pallas_call(
        _log_body,
        grid=(GRID,),
        in_specs=[pl.BlockSpec((NC, N // GRID, B), lambda i: (0, i, 0))],
        out_specs=pl.BlockSpec((N // GRID, B), lambda i: (i, 0)),
        out_shape=jax.ShapeDtypeStruct((N, B), jnp.float32),
    )(partial)
    return out


# speedup vs baseline: 1.6134x; 1.6134x over previous
import jax
import jax.numpy as jnp
from jax import lax
from jax.experimental import pallas as pl
from jax.experimental.pallas import tpu as pltpu
from jax.experimental.pallas import tpu_sc as plsc

N = 10000
B = 128
E = 320000
NC, NS, L = 2, 16, 16
W = NC * NS
BLK = 128
NBLK = E // BLK
BLK_PER_W = -(-NBLK // W)
STRIPE = 624
STRIPE_LAST = N - STRIPE * (NS - 1)
GRID = 10


def _exp_body(x_ref, o_ref):
    o_ref[...] = jnp.exp(x_ref[...])


def _log_body(p_ref, o_ref):
    s = p_ref[0] + p_ref[1]
    o_ref[...] = jnp.log(jnp.maximum(s, 1e-30))


def _sc_body(ev, meta, pf, zeros, out, md_v, p_v, rows_v, s_sh, sem):
    cid = lax.axis_index("c")
    sid = lax.axis_index("s")
    wid = cid * NS + sid
    r0 = sid * STRIPE

    @pl.when(sid < NS - 1)
    def _():
        pltpu.sync_copy(zeros.at[pl.ds(r0, STRIPE)],
                        s_sh.at[pl.ds(r0, STRIPE)])

    @pl.when(sid == NS - 1)
    def _():
        pltpu.sync_copy(zeros.at[pl.ds(r0, STRIPE_LAST)],
                        s_sh.at[pl.ds(r0, STRIPE_LAST)])

    plsc.subcore_barrier()

    def do_block(t, carry):
        blk = wid + t * W

        @pl.when(blk < NBLK)
        def _():
            pltpu.sync_copy(meta.at[blk], md_v)
            pltpu.sync_copy(pf.at[blk], p_v)
            pltpu.async_copy(ev.at[md_v.at[0]], rows_v, sem).wait()

            def mul_group(g, c):
                p16 = p_v[0, pl.ds(g * L, L)]
                for k in range(L):
                    ps = jnp.full((L,), p16[k], jnp.float32)
                    row = g * L + k
                    for j in range(B // L):
                        sl = (row, pl.ds(j * L, L))
                        rows_v[sl] = rows_v[sl] * ps
                return c

            lax.fori_loop(0, BLK // L, mul_group, 0)

            pltpu.sync_copy(rows_v, s_sh.at[md_v.at[1]], add=True)

        return carry

    lax.fori_loop(0, BLK_PER_W, do_block, 0)
    plsc.subcore_barrier()

    @pl.when(sid < NS - 1)
    def _():
        pltpu.sync_copy(s_sh.at[pl.ds(r0, STRIPE)],
                        out.at[cid, pl.ds(r0, STRIPE)])

    @pl.when(sid == NS - 1)
    def _():
        pltpu.sync_copy(s_sh.at[pl.ds(r0, STRIPE_LAST)],
                        out.at[cid, pl.ds(r0, STRIPE_LAST)])


def kernel(ch_vals, edge_src, edge_dst, params):
    ev = pl.pallas_call(
        _exp_body,
        grid=(GRID,),
        in_specs=[pl.BlockSpec((N // GRID, B), lambda i: (i, 0))],
        out_specs=pl.BlockSpec((N // GRID, B), lambda i: (i, 0)),
        out_shape=jax.ShapeDtypeStruct((N, B), jnp.float32),
    )(ch_vals)

    meta = jnp.stack([edge_src.reshape(NBLK, BLK),
                      edge_dst.reshape(NBLK, BLK)], axis=1)
    pf = params.reshape(NBLK, 1, BLK)
    zeros = jnp.zeros((N, B), jnp.float32)

    sc = pl.kernel(
        _sc_body,
        out_type=jax.ShapeDtypeStruct((NC, N, B), jnp.float32),
        mesh=plsc.VectorSubcoreMesh(core_axis_name="c", subcore_axis_name="s"),
        scratch_types=[
            pltpu.VMEM((2, BLK), jnp.int32),
            pltpu.VMEM((1, BLK), jnp.float32),
            pltpu.VMEM((BLK, B), jnp.float32),
            pltpu.VMEM_SHARED((N, B), jnp.float32),
            pltpu.SemaphoreType.DMA,
        ],
    )
    partial = sc(ev, meta, pf, zeros)

    out = pl.pallas_call(
        _log_body,
        grid=(GRID,),
        in_specs=[pl.BlockSpec((NC, N // GRID, B), lambda i: (0, i, 0))],
        out_specs=pl.BlockSpec((N // GRID, B), lambda i: (i, 0)),
        out_shape=jax.ShapeDtypeStruct((N, B), jnp.float32),
    )(partial)
    return out


# speedup vs baseline: 2.2001x; 1.3637x over previous
import jax
import jax.numpy as jnp
from jax import lax
from jax.experimental import pallas as pl
from jax.experimental.pallas import tpu as pltpu
from jax.experimental.pallas import tpu_sc as plsc

N = 10000
B = 128
E = 320000
NC, NS, L = 2, 16, 16
W = NC * NS
BLK = 128
NBLK = E // BLK
BLK_PER_W = -(-NBLK // W)
STRIPE = 624
STRIPE_LAST = N - STRIPE * (NS - 1)
GRID = 10


def _exp_body(x_ref, o_ref):
    o_ref[...] = jnp.exp(x_ref[...])


def _log_body(p_ref, o_ref):
    s = p_ref[0] + p_ref[1]
    o_ref[...] = jnp.log(jnp.maximum(s, 1e-30))


def _sc_body(ev, meta, pf, zeros, out, md_v0, p_v0, rows_v0,
             md_v1, p_v1, rows_v1, s_sh, sem):
    cid = lax.axis_index("c")
    sid = lax.axis_index("s")
    wid = cid * NS + sid
    r0 = sid * STRIPE
    md_v = (md_v0, md_v1)
    p_v = (p_v0, p_v1)
    rows_v = (rows_v0, rows_v1)

    pltpu.sync_copy(meta.at[wid], md_v0)
    pltpu.sync_copy(pf.at[wid], p_v0)
    pltpu.async_copy(ev.at[md_v0.at[0]], rows_v0, sem)

    @pl.when(sid < NS - 1)
    def _():
        pltpu.sync_copy(zeros.at[pl.ds(r0, STRIPE)],
                        s_sh.at[pl.ds(r0, STRIPE)])

    @pl.when(sid == NS - 1)
    def _():
        pltpu.sync_copy(zeros.at[pl.ds(r0, STRIPE_LAST)],
                        s_sh.at[pl.ds(r0, STRIPE_LAST)])

    plsc.subcore_barrier()

    def do_pair(t2, carry):
        for u in range(2):
            t = 2 * t2 + u
            blk = wid + t * W
            mdc, pc, rc = md_v[u], p_v[u], rows_v[u]
            mdn, rn = md_v[1 - u], rows_v[1 - u]

            @pl.when(blk < NBLK)
            def _():
                pltpu.make_async_copy(ev.at[mdc.at[0]], rc, sem).wait()

                @pl.when(blk + W < NBLK)
                def _():
                    pltpu.sync_copy(meta.at[blk + W], mdn)
                    pltpu.sync_copy(pf.at[blk + W], p_v[1 - u])
                    pltpu.async_copy(ev.at[mdn.at[0]], rn, sem)

                def mul_group(g, c):
                    p16 = pc[0, pl.ds(g * L, L)]
                    for k in range(L):
                        ps = jnp.full((L,), p16[k], jnp.float32)
                        row = g * L + k
                        for j in range(B // L):
                            sl = (row, pl.ds(j * L, L))
                            rc[sl] = rc[sl] * ps
                    return c

                lax.fori_loop(0, BLK // L, mul_group, 0)

                pltpu.sync_copy(rc, s_sh.at[mdc.at[1]], add=True)

        return carry

    lax.fori_loop(0, (BLK_PER_W + 1) // 2, do_pair, 0)
    plsc.subcore_barrier()

    @pl.when(sid < NS - 1)
    def _():
        pltpu.sync_copy(s_sh.at[pl.ds(r0, STRIPE)],
                        out.at[cid, pl.ds(r0, STRIPE)])

    @pl.when(sid == NS - 1)
    def _():
        pltpu.sync_copy(s_sh.at[pl.ds(r0, STRIPE_LAST)],
                        out.at[cid, pl.ds(r0, STRIPE_LAST)])


def kernel(ch_vals, edge_src, edge_dst, params):
    ev = pl.pallas_call(
        _exp_body,
        grid=(GRID,),
        in_specs=[pl.BlockSpec((N // GRID, B), lambda i: (i, 0))],
        out_specs=pl.BlockSpec((N // GRID, B), lambda i: (i, 0)),
        out_shape=jax.ShapeDtypeStruct((N, B), jnp.float32),
    )(ch_vals)

    meta = jnp.stack([edge_src.reshape(NBLK, BLK),
                      edge_dst.reshape(NBLK, BLK)], axis=1)
    pf = params.reshape(NBLK, 1, BLK)
    zeros = jnp.zeros((N, B), jnp.float32)

    sc = pl.kernel(
        _sc_body,
        out_type=jax.ShapeDtypeStruct((NC, N, B), jnp.float32),
        mesh=plsc.VectorSubcoreMesh(core_axis_name="c", subcore_axis_name="s"),
        scratch_types=[
            pltpu.VMEM((2, BLK), jnp.int32),
            pltpu.VMEM((1, BLK), jnp.float32),
            pltpu.VMEM((BLK, B), jnp.float32),
            pltpu.VMEM((2, BLK), jnp.int32),
            pltpu.VMEM((1, BLK), jnp.float32),
            pltpu.VMEM((BLK, B), jnp.float32),
            pltpu.VMEM_SHARED((N, B), jnp.float32),
            pltpu.SemaphoreType.DMA,
        ],
    )
    partial = sc(ev, meta, pf, zeros)

    out = pl.pallas_call(
        _log_body,
        grid=(GRID,),
        in_specs=[pl.BlockSpec((NC, N // GRID, B), lambda i: (0, i, 0))],
        out_specs=pl.BlockSpec((N // GRID, B), lambda i: (i, 0)),
        out_shape=jax.ShapeDtypeStruct((N, B), jnp.float32),
    )(partial)
    return out
